# baseline (speedup 1.0000x reference)
import functools

import jax
import jax.numpy as jnp
from jax.experimental import pallas as pl
from jax.experimental.pallas import tpu as pltpu

_EPS = 1e-5
_TAPS = tuple((dy, dx) for dy in range(3) for dx in range(3))


def _scale_shift(st_ref, g_ref, b_ref, count):
    s = st_ref[...]
    inv = 1.0 / count
    mean = jnp.sum(s[:, 0, :], axis=0, keepdims=True) * inv
    ex2 = jnp.sum(s[:, 1, :], axis=0, keepdims=True) * inv
    var = jnp.maximum(ex2 - mean * mean, 0.0)
    scale = jax.lax.rsqrt(var + _EPS) * g_ref[...]
    shift = b_ref[...] - mean * scale
    return scale, shift


def _conv_kernel(*refs, mode, B, H, W, Cin, ct, count, write_aux):
    it = iter(refs)
    x_ref = next(it)
    if mode != "plain":
        st_ref, g_ref, b_ref = next(it), next(it), next(it)
    if mode == "bn_add":
        r_ref = next(it)
    if mode == "bn2":
        ysc_ref, stsc_ref, gsc_ref, bsc_ref = (next(it), next(it), next(it),
                                               next(it))
    w_ref = next(it)
    y_ref = next(it)
    s_ref = next(it)
    aux_ref = next(it) if write_aux else None
    scr = next(it)

    xv = x_ref[...].astype(jnp.float32)
    if mode == "plain":
        hb = xv.astype(jnp.bfloat16)
    else:
        scale, shift = _scale_shift(st_ref, g_ref, b_ref, count)
        h = xv * scale.reshape(1, 1, 1, Cin) + shift.reshape(1, 1, 1, Cin)
        if mode == "bn_add":
            h = h + r_ref[...].astype(jnp.float32)
        elif mode == "bn2":
            sc2, sh2 = _scale_shift(stsc_ref, gsc_ref, bsc_ref, count)
            h = (h + ysc_ref[...].astype(jnp.float32) * sc2.reshape(1, 1, 1, Cin)
                 + sh2.reshape(1, 1, 1, Cin))
        hb = jnp.maximum(h, 0.0).astype(jnp.bfloat16)

    if write_aux:
        @pl.when(pl.program_id(1) == 0)
        def _():
            aux_ref[...] = hb

    @pl.when(pl.program_id(0) == 0)
    def _():
        scr[...] = jnp.zeros(scr.shape, scr.dtype)

    scr[:, 1:H + 1, :, :] = hb

    Hp = H + 2
    Q = B * Hp * W
    v = scr[...].reshape(Q, Cin)
    rowid = jax.lax.broadcasted_iota(jnp.int32, (Q, 1), 0)
    zrow = jnp.zeros((1, Cin), jnp.bfloat16)
    xl = jnp.where(rowid % W == 0, 0,
                   jnp.concatenate([zrow, v[:Q - 1]], axis=0))
    xr = jnp.where(rowid % W == W - 1, 0,
                   jnp.concatenate([v[1:], zrow], axis=0))
    xl4 = xl.reshape(B, Hp, W, Cin)
    xm4 = v.reshape(B, Hp, W, Cin)
    xr4 = xr.reshape(B, Hp, W, Cin)

    acc = None
    for dy in range(3):
        for dx, src in ((0, xl4), (1, xm4), (2, xr4)):
            band = src[:, dy:dy + H, :, :].reshape(B * H * W, Cin)
            d = jnp.dot(band, w_ref[dy * 3 + dx],
                        preferred_element_type=jnp.float32)
            acc = d if acc is None else acc + d
    y_ref[...] = acc.reshape(B, H, W, ct).astype(jnp.bfloat16)
    s_ref[0, 0:1, :] = jnp.sum(acc, axis=0, keepdims=True)
    s_ref[0, 1:2, :] = jnp.sum(acc * acc, axis=0, keepdims=True)


def _conv_call(mode, B, w, x, bn1=None, resid=None, ysc=None, bn2=None,
               write_aux=False):
    N, H, W, Cin = x.shape
    Cout = w.shape[2]
    ct = Cout
    G = N // B
    count = float(N * H * W)

    args = [x]
    in_specs = [pl.BlockSpec((B, H, W, Cin), lambda n, j: (n, 0, 0, 0))]
    if mode != "plain":
        st, g, b = bn1
        args += [st, g, b]
        in_specs += [
            pl.BlockSpec(st.shape, lambda n, j: (0, 0, 0)),
            pl.BlockSpec((1, Cin), lambda n, j: (0, 0)),
            pl.BlockSpec((1, Cin), lambda n, j: (0, 0)),
        ]
    if mode == "bn_add":
        args += [resid]
        in_specs += [pl.BlockSpec((B, H, W, Cin), lambda n, j: (n, 0, 0, 0))]
    if mode == "bn2":
        stsc, gsc, bsc = bn2
        args += [ysc, stsc, gsc, bsc]
        in_specs += [
            pl.BlockSpec((B, H, W, Cin), lambda n, j: (n, 0, 0, 0)),
            pl.BlockSpec(stsc.shape, lambda n, j: (0, 0, 0)),
            pl.BlockSpec((1, Cin), lambda n, j: (0, 0)),
            pl.BlockSpec((1, Cin), lambda n, j: (0, 0)),
        ]
    args.append(w)
    in_specs.append(pl.BlockSpec((9, Cin, ct), lambda n, j: (0, 0, j)))

    out_shape = [jax.ShapeDtypeStruct((N, H, W, Cout), jnp.bfloat16),
                 jax.ShapeDtypeStruct((G, 2, Cout), jnp.float32)]
    out_specs = [pl.BlockSpec((B, H, W, ct), lambda n, j: (n, 0, 0, j)),
                 pl.BlockSpec((1, 2, ct), lambda n, j: (n, 0, j))]
    if write_aux:
        out_shape.append(jax.ShapeDtypeStruct((N, H, W, Cin), jnp.bfloat16))
        out_specs.append(pl.BlockSpec((B, H, W, Cin), lambda n, j: (n, 0, 0, 0)))

    kern = functools.partial(_conv_kernel, mode=mode, B=B, H=H, W=W, Cin=Cin,
                             ct=ct, count=count, write_aux=write_aux)
    return pl.pallas_call(
        kern,
        out_shape=tuple(out_shape),
        grid=(G, Cout // ct),
        in_specs=in_specs,
        out_specs=out_specs,
        scratch_shapes=[pltpu.VMEM((B, H + 2, W, Cin), jnp.bfloat16)],
        compiler_params=pltpu.CompilerParams(
            dimension_semantics=("parallel", "arbitrary")),
    )(*args)


def _sconv_kernel(x_ref, w_ref, wsc_ref, y_ref, s_ref, ysc_ref, ssc_ref, scr,
                  *, B, H, W, Cin, ct):
    Ho, Wo = H // 2, W // 2
    CB = Cin // 128
    scr[:, 1:, CB:, :] = x_ref[...].reshape(B, H, W * CB, 128).astype(
        jnp.float32)

    @pl.when(pl.program_id(0) == 0)
    def _():
        scr[:, 0:1, :, :] = jnp.zeros(scr[:, 0:1, :, :].shape, scr.dtype)
        scr[:, 1:, 0:CB, :] = jnp.zeros(scr[:, 1:, 0:CB, :].shape, scr.dtype)

    acc = None
    asc = None
    for t, (dy, dx) in enumerate(_TAPS):
        for cb in range(CB):
            patch = scr[:, pl.ds(dy, Ho, 2),
                        pl.ds(dx * CB + cb, Wo, 2 * CB), :].reshape(
                B * Ho * Wo, 128).astype(jnp.bfloat16)
            d = jnp.dot(patch, w_ref[t, cb],
                        preferred_element_type=jnp.float32)
            acc = d if acc is None else acc + d
            if (dy, dx) == (1, 1):
                dsc = jnp.dot(patch, wsc_ref[cb],
                              preferred_element_type=jnp.float32)
                asc = dsc if asc is None else asc + dsc
    y_ref[...] = acc.reshape(B, Ho, Wo, ct).astype(jnp.bfloat16)
    s_ref[0, 0:1, :] = jnp.sum(acc, axis=0, keepdims=True)
    s_ref[0, 1:2, :] = jnp.sum(acc * acc, axis=0, keepdims=True)
    ysc_ref[...] = asc.reshape(B, Ho, Wo, ct).astype(jnp.bfloat16)
    ssc_ref[0, 0:1, :] = jnp.sum(asc, axis=0, keepdims=True)
    ssc_ref[0, 1:2, :] = jnp.sum(asc * asc, axis=0, keepdims=True)


def _sconv_call(a, w, wsc, B):
    N, H, W, Cin = a.shape
    Ho, Wo = H // 2, W // 2
    Cout = w.shape[2]
    ct = Cout
    G = N // B
    CB = Cin // 128
    w4 = w.reshape(9, CB, 128, Cout)
    wsc3 = wsc.reshape(CB, 128, Cout)

    kern = functools.partial(_sconv_kernel, B=B, H=H, W=W, Cin=Cin, ct=ct)
    return pl.pallas_call(
        kern,
        out_shape=(jax.ShapeDtypeStruct((N, Ho, Wo, Cout), jnp.bfloat16),
                   jax.ShapeDtypeStruct((G, 2, Cout), jnp.float32),
                   jax.ShapeDtypeStruct((N, Ho, Wo, Cout), jnp.bfloat16),
                   jax.ShapeDtypeStruct((G, 2, Cout), jnp.float32)),
        grid=(G,),
        in_specs=[
            pl.BlockSpec((B, H, W, Cin), lambda n: (n, 0, 0, 0)),
            pl.BlockSpec((9, CB, 128, ct), lambda n: (0, 0, 0, 0)),
            pl.BlockSpec((CB, 128, ct), lambda n: (0, 0, 0)),
        ],
        out_specs=[pl.BlockSpec((B, Ho, Wo, ct), lambda n: (n, 0, 0, 0)),
                   pl.BlockSpec((1, 2, ct), lambda n: (n, 0, 0)),
                   pl.BlockSpec((B, Ho, Wo, ct), lambda n: (n, 0, 0, 0)),
                   pl.BlockSpec((1, 2, ct), lambda n: (n, 0, 0))],
        scratch_shapes=[pltpu.VMEM((B, H + 1, (W + 1) * CB, 128),
                                   jnp.float32)],
        compiler_params=pltpu.CompilerParams(
            dimension_semantics=("arbitrary",)),
    )(a, w4, wsc3)


def _gemm_kernel(x_ref, w_ref, y_ref, s_ref, *, B, M, K, ct):
    acc = jnp.dot(x_ref[...].reshape(B * M, K), w_ref[...],
                  preferred_element_type=jnp.float32)
    y_ref[...] = acc.reshape(B, M, ct).astype(jnp.bfloat16)
    s_ref[0, 0:1, :] = jnp.sum(acc, axis=0, keepdims=True)
    s_ref[0, 1:2, :] = jnp.sum(acc * acc, axis=0, keepdims=True)


def _gemm_call(x3, w2, B):
    N, M, K = x3.shape
    Cout = w2.shape[1]
    ct = Cout
    G = N // B
    kern = functools.partial(_gemm_kernel, B=B, M=M, K=K, ct=ct)
    return pl.pallas_call(
        kern,
        out_shape=(jax.ShapeDtypeStruct((N, M, Cout), jnp.bfloat16),
                   jax.ShapeDtypeStruct((G, 2, Cout), jnp.float32)),
        grid=(G, Cout // ct),
        in_specs=[pl.BlockSpec((B, M, K), lambda n, j: (n, 0, 0)),
                  pl.BlockSpec((K, ct), lambda n, j: (0, j))],
        out_specs=[pl.BlockSpec((B, M, ct), lambda n, j: (n, 0, j)),
                   pl.BlockSpec((1, 2, ct), lambda n, j: (n, 0, j))],
        compiler_params=pltpu.CompilerParams(
            dimension_semantics=("parallel", "arbitrary")),
    )(x3, w2)


def _blend_kernel(y_ref, st_ref, g_ref, b_ref, r_ref, o_ref, *, count, padc):
    scale, shift = _scale_shift(st_ref, g_ref, b_ref, count)
    y = y_ref[...].astype(jnp.float32) * scale + shift
    o = jnp.maximum(y + r_ref[...].astype(jnp.float32), 0.0)
    o_ref[...] = jnp.pad(o, ((0, 0), (0, padc))).astype(o_ref.dtype)


def _blend_pool_kernel(y_ref, st_ref, g_ref, b_ref, r_ref, o_ref, *, count,
                       rows):
    scale, shift = _scale_shift(st_ref, g_ref, b_ref, count)
    y = y_ref[...].astype(jnp.float32) * scale + shift
    o = jnp.maximum(y + r_ref[...].astype(jnp.float32), 0.0)
    ob = o.astype(jnp.bfloat16).astype(jnp.float32)
    tm, tc = y_ref.shape
    o_ref[...] = jnp.mean(ob.reshape(tm // rows, rows, tc), axis=1)


def _blend_call(y2d, st, g, b, r2d, pool_rows=0, pad_to=0):
    M, C = y2d.shape
    Co = max(C, pad_to)
    tm = 4096 if M % 4096 == 0 else (1024 if M % 1024 == 0 else M)
    tc = C
    count = float(M)
    slab = pl.BlockSpec((tm, tc), lambda i, j: (i, j))
    in_specs = [slab,
                pl.BlockSpec(st.shape, lambda i, j: (0, 0, 0)),
                pl.BlockSpec((1, tc), lambda i, j: (0, 0)),
                pl.BlockSpec((1, tc), lambda i, j: (0, 0)),
                slab]
    if pool_rows:
        tm = min(tm, M)
        kern = functools.partial(_blend_pool_kernel, count=count,
                                 rows=pool_rows)
        out_shape = jax.ShapeDtypeStruct((M // pool_rows, C), jnp.float32)
        out_specs = pl.BlockSpec((tm // pool_rows, tc), lambda i, j: (i, j))
    else:
        kern = functools.partial(_blend_kernel, count=count, padc=Co - C)
        out_shape = jax.ShapeDtypeStruct((M, Co), jnp.bfloat16)
        out_specs = pl.BlockSpec((tm, Co), lambda i, j: (i, j))
    return pl.pallas_call(
        kern,
        out_shape=out_shape,
        grid=(M // tm, C // tc),
        in_specs=in_specs,
        out_specs=out_specs,
        compiler_params=pltpu.CompilerParams(
            dimension_semantics=("parallel", "parallel")),
    )(y2d, st, g, b, r2d)


def _im2col_s1(x):
    N, H, W, C = x.shape
    xp = jnp.pad(x, ((0, 0), (1, 1), (1, 1), (0, 0)))
    cols = [xp[:, dy:dy + H, dx:dx + W, :] for dy, dx in _TAPS]
    return jnp.stack(cols, axis=3).reshape(N, H * W, 9 * C)


def kernel(x, conv1_w, bn1_g, bn1_b, layer1_0_conv1_w, layer1_0_conv2_w, layer1_0_bn1_g, layer1_0_bn1_b, layer1_0_bn2_g, layer1_0_bn2_b, layer1_1_conv1_w, layer1_1_conv2_w, layer1_1_bn1_g, layer1_1_bn1_b, layer1_1_bn2_g, layer1_1_bn2_b, layer2_0_conv1_w, layer2_0_conv2_w, layer2_0_bn1_g, layer2_0_bn1_b, layer2_0_bn2_g, layer2_0_bn2_b, layer2_0_sc_w, layer2_0_sc_g, layer2_0_sc_b, layer2_1_conv1_w, layer2_1_conv2_w, layer2_1_bn1_g, layer2_1_bn1_b, layer2_1_bn2_g, layer2_1_bn2_b, layer3_0_conv1_w, layer3_0_conv2_w, layer3_0_bn1_g, layer3_0_bn1_b, layer3_0_bn2_g, layer3_0_bn2_b, layer3_0_sc_w, layer3_0_sc_g, layer3_0_sc_b, layer3_1_conv1_w, layer3_1_conv2_w, layer3_1_bn1_g, layer3_1_bn1_b, layer3_1_bn2_g, layer3_1_bn2_b, layer4_0_conv1_w, layer4_0_conv2_w, layer4_0_bn1_g, layer4_0_bn1_b, layer4_0_bn2_g, layer4_0_bn2_b, layer4_0_sc_w, layer4_0_sc_g, layer4_0_sc_b, layer4_1_conv1_w, layer4_1_conv2_w, layer4_1_bn1_g, layer4_1_bn1_b, layer4_1_bn2_g, layer4_1_bn2_b, fc_w, fc_b):
    N = x.shape[0]
    xb = jnp.transpose(x, (0, 2, 3, 1)).astype(jnp.bfloat16)

    cols = _im2col_s1(xb)
    Cr = 64 if conv1_w.shape[2] == 128 else conv1_w.shape[2]
    y0, s0 = _gemm_call(cols, conv1_w.reshape(-1, conv1_w.shape[2])[:, :Cr],
                        B=min(4, N))
    y0 = y0.reshape(N, 32, 32, Cr)

    def cw(w):
        return w[:, :Cr, :Cr]

    def cv(v):
        return v[:, :Cr]

    y1, s1, h0 = _conv_call("bn", min(4, N), cw(layer1_0_conv1_w), y0,
                            bn1=(s0, cv(bn1_g), cv(bn1_b)), write_aux=True)
    y2, s2 = _conv_call("bn", min(4, N), cw(layer1_0_conv2_w), y1,
                        bn1=(s1, cv(layer1_0_bn1_g), cv(layer1_0_bn1_b)))
    y1b, s1b, a10 = _conv_call("bn_add", min(4, N), cw(layer1_1_conv1_w), y2,
                               bn1=(s2, cv(layer1_0_bn2_g),
                                    cv(layer1_0_bn2_b)),
                               resid=h0, write_aux=True)
    y2b, s2b = _conv_call("bn", min(4, N), cw(layer1_1_conv2_w), y1b,
                          bn1=(s1b, cv(layer1_1_bn1_g), cv(layer1_1_bn1_b)))
    C1 = layer1_1_conv2_w.shape[2]
    a1 = _blend_call(y2b.reshape(-1, Cr), s2b, cv(layer1_1_bn2_g),
                     cv(layer1_1_bn2_b), a10.reshape(-1, Cr),
                     pad_to=C1).reshape(N, 32, 32, C1)

    def down_stage(a, c1w, c2w, g1, b1, g2, b2, scw, scg, scb,
                   c1wB, c2wB, g1B, b1B, g2B, b2B, B, Bs, pool_rows=0):
        Ho = a.shape[1] // 2
        Cout = c1w.shape[2]
        y1, s1, ysc, ssc = _sconv_call(a, c1w, scw, Bs)
        y2, s2 = _conv_call("bn", B, c2w, y1, bn1=(s1, g1, b1))
        y1b, s1b, a0 = _conv_call("bn2", B, c1wB, y2, bn1=(s2, g2, b2),
                                  ysc=ysc, bn2=(ssc, scg, scb),
                                  write_aux=True)
        y2b, s2b = _conv_call("bn", B, c2wB, y1b, bn1=(s1b, g1B, b1B))
        out = _blend_call(y2b.reshape(-1, Cout), s2b, g2B, b2B,
                          a0.reshape(-1, Cout), pool_rows=pool_rows)
        if pool_rows:
            return out
        return out.reshape(N, Ho, Ho, Cout)

    a2 = down_stage(a1, layer2_0_conv1_w, layer2_0_conv2_w,
                    layer2_0_bn1_g, layer2_0_bn1_b, layer2_0_bn2_g,
                    layer2_0_bn2_b, layer2_0_sc_w, layer2_0_sc_g,
                    layer2_0_sc_b, layer2_1_conv1_w, layer2_1_conv2_w,
                    layer2_1_bn1_g, layer2_1_bn1_b, layer2_1_bn2_g,
                    layer2_1_bn2_b, B=min(8, N), Bs=min(4, N))
    a3 = down_stage(a2, layer3_0_conv1_w, layer3_0_conv2_w,
                    layer3_0_bn1_g, layer3_0_bn1_b, layer3_0_bn2_g,
                    layer3_0_bn2_b, layer3_0_sc_w, layer3_0_sc_g,
                    layer3_0_sc_b, layer3_1_conv1_w, layer3_1_conv2_w,
                    layer3_1_bn1_g, layer3_1_bn1_b, layer3_1_bn2_g,
                    layer3_1_bn2_b, B=min(16, N), Bs=min(8, N))
    pooled = down_stage(a3, layer4_0_conv1_w, layer4_0_conv2_w,
                        layer4_0_bn1_g, layer4_0_bn1_b, layer4_0_bn2_g,
                        layer4_0_bn2_b, layer4_0_sc_w, layer4_0_sc_g,
                        layer4_0_sc_b, layer4_1_conv1_w, layer4_1_conv2_w,
                        layer4_1_bn1_g, layer4_1_bn1_b, layer4_1_bn2_g,
                        layer4_1_bn2_b, B=min(32, N), Bs=min(16, N), pool_rows=16)

    return pooled @ fc_w + fc_b


# speedup vs baseline: 1.0150x; 1.0150x over previous
import functools

import jax
import jax.numpy as jnp
from jax.experimental import pallas as pl
from jax.experimental.pallas import tpu as pltpu

_EPS = 1e-5
_TAPS = tuple((dy, dx) for dy in range(3) for dx in range(3))


def _scale_shift(st_ref, g_ref, b_ref, count):
    s = st_ref[...]
    inv = 1.0 / count
    mean = jnp.sum(s[:, 0, :], axis=0, keepdims=True) * inv
    ex2 = jnp.sum(s[:, 1, :], axis=0, keepdims=True) * inv
    var = jnp.maximum(ex2 - mean * mean, 0.0)
    scale = jax.lax.rsqrt(var + _EPS) * g_ref[...]
    shift = b_ref[...] - mean * scale
    return scale, shift


def _conv_kernel(*refs, mode, B, H, W, Cin, ct, count, write_aux):
    it = iter(refs)
    x_ref = next(it)
    if mode != "plain":
        st_ref, g_ref, b_ref = next(it), next(it), next(it)
    if mode == "bn_add":
        r_ref = next(it)
    if mode == "bn2":
        ysc_ref, stsc_ref, gsc_ref, bsc_ref = (next(it), next(it), next(it),
                                               next(it))
    w_ref = next(it)
    y_ref = next(it)
    s_ref = next(it)
    aux_ref = next(it) if write_aux else None
    scr = next(it)

    xv = x_ref[...].astype(jnp.float32)
    if mode == "plain":
        hb = xv.astype(jnp.bfloat16)
    else:
        scale, shift = _scale_shift(st_ref, g_ref, b_ref, count)
        h = xv * scale.reshape(1, 1, 1, Cin) + shift.reshape(1, 1, 1, Cin)
        if mode == "bn_add":
            h = h + r_ref[...].astype(jnp.float32)
        elif mode == "bn2":
            sc2, sh2 = _scale_shift(stsc_ref, gsc_ref, bsc_ref, count)
            h = (h + ysc_ref[...].astype(jnp.float32) * sc2.reshape(1, 1, 1, Cin)
                 + sh2.reshape(1, 1, 1, Cin))
        hb = jnp.maximum(h, 0.0).astype(jnp.bfloat16)

    if write_aux:
        @pl.when(pl.program_id(1) == 0)
        def _():
            aux_ref[...] = hb

    @pl.when(pl.program_id(0) == 0)
    def _():
        scr[...] = jnp.zeros(scr.shape, scr.dtype)

    scr[:, 1:H + 1, :, :] = hb

    Hp = H + 2
    Q = B * Hp * W
    v = scr[...].reshape(Q, Cin)
    rowid = jax.lax.broadcasted_iota(jnp.int32, (Q, 1), 0)
    zrow = jnp.zeros((1, Cin), jnp.bfloat16)
    xl = jnp.where(rowid % W == 0, 0,
                   jnp.concatenate([zrow, v[:Q - 1]], axis=0))
    xr = jnp.where(rowid % W == W - 1, 0,
                   jnp.concatenate([v[1:], zrow], axis=0))
    xl4 = xl.reshape(B, Hp, W, Cin)
    xm4 = v.reshape(B, Hp, W, Cin)
    xr4 = xr.reshape(B, Hp, W, Cin)

    acc = None
    for dy in range(3):
        for dx, src in ((0, xl4), (1, xm4), (2, xr4)):
            band = src[:, dy:dy + H, :, :].reshape(B * H * W, Cin)
            d = jnp.dot(band, w_ref[dy * 3 + dx],
                        preferred_element_type=jnp.float32)
            acc = d if acc is None else acc + d
    y_ref[...] = acc.reshape(B, H, W, ct).astype(jnp.bfloat16)
    s_ref[0, 0:1, :] = jnp.sum(acc, axis=0, keepdims=True)
    s_ref[0, 1:2, :] = jnp.sum(acc * acc, axis=0, keepdims=True)


def _conv_call(mode, B, w, x, bn1=None, resid=None, ysc=None, bn2=None,
               write_aux=False):
    N, H, W, Cin = x.shape
    Cout = w.shape[2]
    ct = Cout
    G = N // B
    count = float(N * H * W)

    args = [x]
    in_specs = [pl.BlockSpec((B, H, W, Cin), lambda n, j: (n, 0, 0, 0))]
    if mode != "plain":
        st, g, b = bn1
        args += [st, g, b]
        in_specs += [
            pl.BlockSpec(st.shape, lambda n, j: (0, 0, 0)),
            pl.BlockSpec((1, Cin), lambda n, j: (0, 0)),
            pl.BlockSpec((1, Cin), lambda n, j: (0, 0)),
        ]
    if mode == "bn_add":
        args += [resid]
        in_specs += [pl.BlockSpec((B, H, W, Cin), lambda n, j: (n, 0, 0, 0))]
    if mode == "bn2":
        stsc, gsc, bsc = bn2
        args += [ysc, stsc, gsc, bsc]
        in_specs += [
            pl.BlockSpec((B, H, W, Cin), lambda n, j: (n, 0, 0, 0)),
            pl.BlockSpec(stsc.shape, lambda n, j: (0, 0, 0)),
            pl.BlockSpec((1, Cin), lambda n, j: (0, 0)),
            pl.BlockSpec((1, Cin), lambda n, j: (0, 0)),
        ]
    args.append(w)
    in_specs.append(pl.BlockSpec((9, Cin, ct), lambda n, j: (0, 0, j)))

    out_shape = [jax.ShapeDtypeStruct((N, H, W, Cout), jnp.bfloat16),
                 jax.ShapeDtypeStruct((G, 2, Cout), jnp.float32)]
    out_specs = [pl.BlockSpec((B, H, W, ct), lambda n, j: (n, 0, 0, j)),
                 pl.BlockSpec((1, 2, ct), lambda n, j: (n, 0, j))]
    if write_aux:
        out_shape.append(jax.ShapeDtypeStruct((N, H, W, Cin), jnp.bfloat16))
        out_specs.append(pl.BlockSpec((B, H, W, Cin), lambda n, j: (n, 0, 0, 0)))

    kern = functools.partial(_conv_kernel, mode=mode, B=B, H=H, W=W, Cin=Cin,
                             ct=ct, count=count, write_aux=write_aux)
    return pl.pallas_call(
        kern,
        out_shape=tuple(out_shape),
        grid=(G, Cout // ct),
        in_specs=in_specs,
        out_specs=out_specs,
        scratch_shapes=[pltpu.VMEM((B, H + 2, W, Cin), jnp.bfloat16)],
        compiler_params=pltpu.CompilerParams(
            dimension_semantics=("parallel", "arbitrary")),
    )(*args)


def _sconv_kernel(x_ref, w_ref, wsc_ref, y_ref, s_ref, ysc_ref, ssc_ref, scr,
                  *, B, H, W, Cin, ct):
    Ho, Wo = H // 2, W // 2
    CB = Cin // 128
    scr[:, 1:, CB:, :] = x_ref[...].reshape(B, H, W * CB, 128).astype(
        jnp.float32)

    @pl.when(pl.program_id(0) == 0)
    def _():
        scr[:, 0:1, :, :] = jnp.zeros(scr[:, 0:1, :, :].shape, scr.dtype)
        scr[:, 1:, 0:CB, :] = jnp.zeros(scr[:, 1:, 0:CB, :].shape, scr.dtype)

    acc = None
    asc = None
    for t, (dy, dx) in enumerate(_TAPS):
        for cb in range(CB):
            patch = scr[:, pl.ds(dy, Ho, 2),
                        pl.ds(dx * CB + cb, Wo, 2 * CB), :].reshape(
                B * Ho * Wo, 128).astype(jnp.bfloat16)
            d = jnp.dot(patch, w_ref[t, cb],
                        preferred_element_type=jnp.float32)
            acc = d if acc is None else acc + d
            if (dy, dx) == (1, 1):
                dsc = jnp.dot(patch, wsc_ref[cb],
                              preferred_element_type=jnp.float32)
                asc = dsc if asc is None else asc + dsc
    y_ref[...] = acc.reshape(B, Ho, Wo, ct).astype(jnp.bfloat16)
    s_ref[0, 0:1, :] = jnp.sum(acc, axis=0, keepdims=True)
    s_ref[0, 1:2, :] = jnp.sum(acc * acc, axis=0, keepdims=True)
    ysc_ref[...] = asc.reshape(B, Ho, Wo, ct).astype(jnp.bfloat16)
    ssc_ref[0, 0:1, :] = jnp.sum(asc, axis=0, keepdims=True)
    ssc_ref[0, 1:2, :] = jnp.sum(asc * asc, axis=0, keepdims=True)


def _sconv_call(a, w, wsc, B):
    N, H, W, Cin = a.shape
    Ho, Wo = H // 2, W // 2
    Cout = w.shape[2]
    ct = Cout
    G = N // B
    CB = Cin // 128
    w4 = w.reshape(9, CB, 128, Cout)
    wsc3 = wsc.reshape(CB, 128, Cout)

    kern = functools.partial(_sconv_kernel, B=B, H=H, W=W, Cin=Cin, ct=ct)
    return pl.pallas_call(
        kern,
        out_shape=(jax.ShapeDtypeStruct((N, Ho, Wo, Cout), jnp.bfloat16),
                   jax.ShapeDtypeStruct((G, 2, Cout), jnp.float32),
                   jax.ShapeDtypeStruct((N, Ho, Wo, Cout), jnp.bfloat16),
                   jax.ShapeDtypeStruct((G, 2, Cout), jnp.float32)),
        grid=(G,),
        in_specs=[
            pl.BlockSpec((B, H, W, Cin), lambda n: (n, 0, 0, 0)),
            pl.BlockSpec((9, CB, 128, ct), lambda n: (0, 0, 0, 0)),
            pl.BlockSpec((CB, 128, ct), lambda n: (0, 0, 0)),
        ],
        out_specs=[pl.BlockSpec((B, Ho, Wo, ct), lambda n: (n, 0, 0, 0)),
                   pl.BlockSpec((1, 2, ct), lambda n: (n, 0, 0)),
                   pl.BlockSpec((B, Ho, Wo, ct), lambda n: (n, 0, 0, 0)),
                   pl.BlockSpec((1, 2, ct), lambda n: (n, 0, 0))],
        scratch_shapes=[pltpu.VMEM((B, H + 1, (W + 1) * CB, 128),
                                   jnp.float32)],
        compiler_params=pltpu.CompilerParams(
            dimension_semantics=("arbitrary",)),
    )(a, w4, wsc3)


def _gemm_kernel(x_ref, w_ref, y_ref, s_ref, *, B, M, K, ct):
    acc = jnp.dot(x_ref[...].reshape(B * M, K), w_ref[...],
                  preferred_element_type=jnp.float32)
    y_ref[...] = acc.reshape(B, M, ct).astype(jnp.bfloat16)
    s_ref[0, 0:1, :] = jnp.sum(acc, axis=0, keepdims=True)
    s_ref[0, 1:2, :] = jnp.sum(acc * acc, axis=0, keepdims=True)


def _gemm_call(x3, w2, B):
    N, M, K = x3.shape
    Cout = w2.shape[1]
    ct = Cout
    G = N // B
    kern = functools.partial(_gemm_kernel, B=B, M=M, K=K, ct=ct)
    return pl.pallas_call(
        kern,
        out_shape=(jax.ShapeDtypeStruct((N, M, Cout), jnp.bfloat16),
                   jax.ShapeDtypeStruct((G, 2, Cout), jnp.float32)),
        grid=(G, Cout // ct),
        in_specs=[pl.BlockSpec((B, M, K), lambda n, j: (n, 0, 0)),
                  pl.BlockSpec((K, ct), lambda n, j: (0, j))],
        out_specs=[pl.BlockSpec((B, M, ct), lambda n, j: (n, 0, j)),
                   pl.BlockSpec((1, 2, ct), lambda n, j: (n, 0, j))],
        compiler_params=pltpu.CompilerParams(
            dimension_semantics=("parallel", "arbitrary")),
    )(x3, w2)


def _blend_kernel(y_ref, st_ref, g_ref, b_ref, r_ref, o_ref, *, count, padc):
    scale, shift = _scale_shift(st_ref, g_ref, b_ref, count)
    y = y_ref[...].astype(jnp.float32) * scale + shift
    o = jnp.maximum(y + r_ref[...].astype(jnp.float32), 0.0)
    o_ref[...] = jnp.pad(o, ((0, 0), (0, padc))).astype(o_ref.dtype)


def _blend_pool_kernel(y_ref, st_ref, g_ref, b_ref, r_ref, o_ref, *, count,
                       rows):
    scale, shift = _scale_shift(st_ref, g_ref, b_ref, count)
    y = y_ref[...].astype(jnp.float32) * scale + shift
    o = jnp.maximum(y + r_ref[...].astype(jnp.float32), 0.0)
    ob = o.astype(jnp.bfloat16).astype(jnp.float32)
    tm, tc = y_ref.shape
    o_ref[...] = jnp.mean(ob.reshape(tm // rows, rows, tc), axis=1)


def _blend_call(y2d, st, g, b, r2d, pool_rows=0, pad_to=0):
    M, C = y2d.shape
    Co = max(C, pad_to)
    tm = 4096 if M % 4096 == 0 else (1024 if M % 1024 == 0 else M)
    tc = C
    count = float(M)
    slab = pl.BlockSpec((tm, tc), lambda i, j: (i, j))
    in_specs = [slab,
                pl.BlockSpec(st.shape, lambda i, j: (0, 0, 0)),
                pl.BlockSpec((1, tc), lambda i, j: (0, 0)),
                pl.BlockSpec((1, tc), lambda i, j: (0, 0)),
                slab]
    if pool_rows:
        tm = min(tm, M)
        kern = functools.partial(_blend_pool_kernel, count=count,
                                 rows=pool_rows)
        out_shape = jax.ShapeDtypeStruct((M // pool_rows, C), jnp.float32)
        out_specs = pl.BlockSpec((tm // pool_rows, tc), lambda i, j: (i, j))
    else:
        kern = functools.partial(_blend_kernel, count=count, padc=Co - C)
        out_shape = jax.ShapeDtypeStruct((M, Co), jnp.bfloat16)
        out_specs = pl.BlockSpec((tm, Co), lambda i, j: (i, j))
    return pl.pallas_call(
        kern,
        out_shape=out_shape,
        grid=(M // tm, C // tc),
        in_specs=in_specs,
        out_specs=out_specs,
        compiler_params=pltpu.CompilerParams(
            dimension_semantics=("parallel", "parallel")),
    )(y2d, st, g, b, r2d)


def _im2col_s1(x):
    N, H, W, C = x.shape
    xp = jnp.pad(x, ((0, 0), (1, 1), (1, 1), (0, 0)))
    cols = [xp[:, dy:dy + H, dx:dx + W, :] for dy, dx in _TAPS]
    return jnp.stack(cols, axis=3).reshape(N, H * W, 9 * C)


def kernel(x, conv1_w, bn1_g, bn1_b, layer1_0_conv1_w, layer1_0_conv2_w, layer1_0_bn1_g, layer1_0_bn1_b, layer1_0_bn2_g, layer1_0_bn2_b, layer1_1_conv1_w, layer1_1_conv2_w, layer1_1_bn1_g, layer1_1_bn1_b, layer1_1_bn2_g, layer1_1_bn2_b, layer2_0_conv1_w, layer2_0_conv2_w, layer2_0_bn1_g, layer2_0_bn1_b, layer2_0_bn2_g, layer2_0_bn2_b, layer2_0_sc_w, layer2_0_sc_g, layer2_0_sc_b, layer2_1_conv1_w, layer2_1_conv2_w, layer2_1_bn1_g, layer2_1_bn1_b, layer2_1_bn2_g, layer2_1_bn2_b, layer3_0_conv1_w, layer3_0_conv2_w, layer3_0_bn1_g, layer3_0_bn1_b, layer3_0_bn2_g, layer3_0_bn2_b, layer3_0_sc_w, layer3_0_sc_g, layer3_0_sc_b, layer3_1_conv1_w, layer3_1_conv2_w, layer3_1_bn1_g, layer3_1_bn1_b, layer3_1_bn2_g, layer3_1_bn2_b, layer4_0_conv1_w, layer4_0_conv2_w, layer4_0_bn1_g, layer4_0_bn1_b, layer4_0_bn2_g, layer4_0_bn2_b, layer4_0_sc_w, layer4_0_sc_g, layer4_0_sc_b, layer4_1_conv1_w, layer4_1_conv2_w, layer4_1_bn1_g, layer4_1_bn1_b, layer4_1_bn2_g, layer4_1_bn2_b, fc_w, fc_b):
    N = x.shape[0]
    xb = jnp.transpose(x, (0, 2, 3, 1)).astype(jnp.bfloat16)

    cols = _im2col_s1(xb)
    Cr = conv1_w.shape[2]
    y0, s0 = _gemm_call(cols, conv1_w.reshape(-1, conv1_w.shape[2])[:, :Cr],
                        B=min(4, N))
    y0 = y0.reshape(N, 32, 32, Cr)

    def cw(w):
        return w[:, :Cr, :Cr]

    def cv(v):
        return v[:, :Cr]

    y1, s1, h0 = _conv_call("bn", min(4, N), cw(layer1_0_conv1_w), y0,
                            bn1=(s0, cv(bn1_g), cv(bn1_b)), write_aux=True)
    y2, s2 = _conv_call("bn", min(4, N), cw(layer1_0_conv2_w), y1,
                        bn1=(s1, cv(layer1_0_bn1_g), cv(layer1_0_bn1_b)))
    y1b, s1b, a10 = _conv_call("bn_add", min(4, N), cw(layer1_1_conv1_w), y2,
                               bn1=(s2, cv(layer1_0_bn2_g),
                                    cv(layer1_0_bn2_b)),
                               resid=h0, write_aux=True)
    y2b, s2b = _conv_call("bn", min(4, N), cw(layer1_1_conv2_w), y1b,
                          bn1=(s1b, cv(layer1_1_bn1_g), cv(layer1_1_bn1_b)))
    C1 = layer1_1_conv2_w.shape[2]
    a1 = _blend_call(y2b.reshape(-1, Cr), s2b, cv(layer1_1_bn2_g),
                     cv(layer1_1_bn2_b), a10.reshape(-1, Cr),
                     pad_to=C1).reshape(N, 32, 32, C1)

    def down_stage(a, c1w, c2w, g1, b1, g2, b2, scw, scg, scb,
                   c1wB, c2wB, g1B, b1B, g2B, b2B, B, Bs, pool_rows=0):
        Ho = a.shape[1] // 2
        Cout = c1w.shape[2]
        y1, s1, ysc, ssc = _sconv_call(a, c1w, scw, Bs)
        y2, s2 = _conv_call("bn", B, c2w, y1, bn1=(s1, g1, b1))
        y1b, s1b, a0 = _conv_call("bn2", B, c1wB, y2, bn1=(s2, g2, b2),
                                  ysc=ysc, bn2=(ssc, scg, scb),
                                  write_aux=True)
        y2b, s2b = _conv_call("bn", B, c2wB, y1b, bn1=(s1b, g1B, b1B))
        out = _blend_call(y2b.reshape(-1, Cout), s2b, g2B, b2B,
                          a0.reshape(-1, Cout), pool_rows=pool_rows)
        if pool_rows:
            return out
        return out.reshape(N, Ho, Ho, Cout)

    a2 = down_stage(a1, layer2_0_conv1_w, layer2_0_conv2_w,
                    layer2_0_bn1_g, layer2_0_bn1_b, layer2_0_bn2_g,
                    layer2_0_bn2_b, layer2_0_sc_w, layer2_0_sc_g,
                    layer2_0_sc_b, layer2_1_conv1_w, layer2_1_conv2_w,
                    layer2_1_bn1_g, layer2_1_bn1_b, layer2_1_bn2_g,
                    layer2_1_bn2_b, B=min(8, N), Bs=min(4, N))
    a3 = down_stage(a2, layer3_0_conv1_w, layer3_0_conv2_w,
                    layer3_0_bn1_g, layer3_0_bn1_b, layer3_0_bn2_g,
                    layer3_0_bn2_b, layer3_0_sc_w, layer3_0_sc_g,
                    layer3_0_sc_b, layer3_1_conv1_w, layer3_1_conv2_w,
                    layer3_1_bn1_g, layer3_1_bn1_b, layer3_1_bn2_g,
                    layer3_1_bn2_b, B=min(16, N), Bs=min(8, N))
    pooled = down_stage(a3, layer4_0_conv1_w, layer4_0_conv2_w,
                        layer4_0_bn1_g, layer4_0_bn1_b, layer4_0_bn2_g,
                        layer4_0_bn2_b, layer4_0_sc_w, layer4_0_sc_g,
                        layer4_0_sc_b, layer4_1_conv1_w, layer4_1_conv2_w,
                        layer4_1_bn1_g, layer4_1_bn1_b, layer4_1_bn2_g,
                        layer4_1_bn2_b, B=min(32, N), Bs=min(16, N), pool_rows=16)

    return pooled @ fc_w + fc_b


# speedup vs baseline: 1.0278x; 1.0127x over previous
import functools

import jax
import jax.numpy as jnp
from jax.experimental import pallas as pl
from jax.experimental.pallas import tpu as pltpu

_EPS = 1e-5
_TAPS = tuple((dy, dx) for dy in range(3) for dx in range(3))


def _scale_shift(st_ref, g_ref, b_ref, count):
    s = st_ref[...]
    inv = 1.0 / count
    mean = jnp.sum(s[:, 0, :], axis=0, keepdims=True) * inv
    ex2 = jnp.sum(s[:, 1, :], axis=0, keepdims=True) * inv
    var = jnp.maximum(ex2 - mean * mean, 0.0)
    scale = jax.lax.rsqrt(var + _EPS) * g_ref[...]
    shift = b_ref[...] - mean * scale
    return scale, shift


def _conv_kernel(*refs, mode, B, H, W, Cin, ct, count, write_aux):
    it = iter(refs)
    x_ref = next(it)
    if mode != "plain":
        st_ref, g_ref, b_ref = next(it), next(it), next(it)
    if mode == "bn_add":
        r_ref = next(it)
    if mode == "bn2":
        ysc_ref, stsc_ref, gsc_ref, bsc_ref = (next(it), next(it), next(it),
                                               next(it))
    w_ref = next(it)
    y_ref = next(it)
    s_ref = next(it)
    aux_ref = next(it) if write_aux else None
    scr = next(it)

    xv = x_ref[...].astype(jnp.float32)
    if mode == "plain":
        hb = xv.astype(jnp.bfloat16)
    else:
        scale, shift = _scale_shift(st_ref, g_ref, b_ref, count)
        h = xv * scale.reshape(1, 1, 1, Cin) + shift.reshape(1, 1, 1, Cin)
        if mode == "bn_add":
            h = h + r_ref[...].astype(jnp.float32)
        elif mode == "bn2":
            sc2, sh2 = _scale_shift(stsc_ref, gsc_ref, bsc_ref, count)
            h = (h + ysc_ref[...].astype(jnp.float32) * sc2.reshape(1, 1, 1, Cin)
                 + sh2.reshape(1, 1, 1, Cin))
        hb = jnp.maximum(h, 0.0).astype(jnp.bfloat16)

    if write_aux:
        @pl.when(pl.program_id(1) == 0)
        def _():
            aux_ref[...] = hb

    @pl.when(pl.program_id(0) == 0)
    def _():
        scr[...] = jnp.zeros(scr.shape, scr.dtype)

    scr[:, 1:H + 1, :, :] = hb

    Hp = H + 2
    Q = B * Hp * W
    v = scr[...].reshape(Q, Cin)
    rowid = jax.lax.broadcasted_iota(jnp.int32, (Q, 1), 0)
    zrow = jnp.zeros((1, Cin), jnp.bfloat16)
    xl = jnp.where(rowid % W == 0, 0,
                   jnp.concatenate([zrow, v[:Q - 1]], axis=0))
    xr = jnp.where(rowid % W == W - 1, 0,
                   jnp.concatenate([v[1:], zrow], axis=0))
    xl4 = xl.reshape(B, Hp, W, Cin)
    xm4 = v.reshape(B, Hp, W, Cin)
    xr4 = xr.reshape(B, Hp, W, Cin)

    acc = None
    if W % 8 == 0:
        for dx, src in ((0, xl4), (1, xm4), (2, xr4)):
            wide = jnp.concatenate(
                [src[:, dy:dy + H, :, :] for dy in range(3)],
                axis=3).reshape(B * H * W, 3 * Cin)
            d = jnp.dot(wide, w_ref[dx], preferred_element_type=jnp.float32)
            acc = d if acc is None else acc + d
    else:
        for dy in range(3):
            for dx, src in ((0, xl4), (1, xm4), (2, xr4)):
                band = src[:, dy:dy + H, :, :].reshape(B * H * W, Cin)
                d = jnp.dot(band, w_ref[dy * 3 + dx],
                            preferred_element_type=jnp.float32)
                acc = d if acc is None else acc + d
    y_ref[...] = acc.reshape(B, H, W, ct).astype(jnp.bfloat16)
    s_ref[0, 0:1, :] = jnp.sum(acc, axis=0, keepdims=True)
    s_ref[0, 1:2, :] = jnp.sum(acc * acc, axis=0, keepdims=True)


def _conv_call(mode, B, w, x, bn1=None, resid=None, ysc=None, bn2=None,
               write_aux=False):
    N, H, W, Cin = x.shape
    Cout = w.shape[2]
    ct = Cout
    G = N // B
    count = float(N * H * W)

    args = [x]
    in_specs = [pl.BlockSpec((B, H, W, Cin), lambda n, j: (n, 0, 0, 0))]
    if mode != "plain":
        st, g, b = bn1
        args += [st, g, b]
        in_specs += [
            pl.BlockSpec(st.shape, lambda n, j: (0, 0, 0)),
            pl.BlockSpec((1, Cin), lambda n, j: (0, 0)),
            pl.BlockSpec((1, Cin), lambda n, j: (0, 0)),
        ]
    if mode == "bn_add":
        args += [resid]
        in_specs += [pl.BlockSpec((B, H, W, Cin), lambda n, j: (n, 0, 0, 0))]
    if mode == "bn2":
        stsc, gsc, bsc = bn2
        args += [ysc, stsc, gsc, bsc]
        in_specs += [
            pl.BlockSpec((B, H, W, Cin), lambda n, j: (n, 0, 0, 0)),
            pl.BlockSpec(stsc.shape, lambda n, j: (0, 0, 0)),
            pl.BlockSpec((1, Cin), lambda n, j: (0, 0)),
            pl.BlockSpec((1, Cin), lambda n, j: (0, 0)),
        ]
    if W % 8 == 0:
        w = jnp.stack([jnp.concatenate([w[dy * 3 + dx] for dy in range(3)],
                                       axis=0) for dx in range(3)])
        args.append(w)
        in_specs.append(pl.BlockSpec((3, 3 * Cin, ct), lambda n, j: (0, 0, j)))
    else:
        args.append(w)
        in_specs.append(pl.BlockSpec((9, Cin, ct), lambda n, j: (0, 0, j)))

    out_shape = [jax.ShapeDtypeStruct((N, H, W, Cout), jnp.bfloat16),
                 jax.ShapeDtypeStruct((G, 2, Cout), jnp.float32)]
    out_specs = [pl.BlockSpec((B, H, W, ct), lambda n, j: (n, 0, 0, j)),
                 pl.BlockSpec((1, 2, ct), lambda n, j: (n, 0, j))]
    if write_aux:
        out_shape.append(jax.ShapeDtypeStruct((N, H, W, Cin), jnp.bfloat16))
        out_specs.append(pl.BlockSpec((B, H, W, Cin), lambda n, j: (n, 0, 0, 0)))

    kern = functools.partial(_conv_kernel, mode=mode, B=B, H=H, W=W, Cin=Cin,
                             ct=ct, count=count, write_aux=write_aux)
    return pl.pallas_call(
        kern,
        out_shape=tuple(out_shape),
        grid=(G, Cout // ct),
        in_specs=in_specs,
        out_specs=out_specs,
        scratch_shapes=[pltpu.VMEM((B, H + 2, W, Cin), jnp.bfloat16)],
        compiler_params=pltpu.CompilerParams(
            dimension_semantics=("parallel", "arbitrary")),
    )(*args)


def _sconv_kernel(x_ref, w_ref, wsc_ref, y_ref, s_ref, ysc_ref, ssc_ref, scr,
                  *, B, H, W, Cin, ct):
    Ho, Wo = H // 2, W // 2
    CB = Cin // 128
    scr[:, 1:, CB:, :] = x_ref[...].reshape(B, H, W * CB, 128).astype(
        jnp.float32)

    @pl.when(pl.program_id(0) == 0)
    def _():
        scr[:, 0:1, :, :] = jnp.zeros(scr[:, 0:1, :, :].shape, scr.dtype)
        scr[:, 1:, 0:CB, :] = jnp.zeros(scr[:, 1:, 0:CB, :].shape, scr.dtype)

    acc = None
    asc = None
    for t, (dy, dx) in enumerate(_TAPS):
        for cb in range(CB):
            patch = scr[:, pl.ds(dy, Ho, 2),
                        pl.ds(dx * CB + cb, Wo, 2 * CB), :].reshape(
                B * Ho * Wo, 128).astype(jnp.bfloat16)
            d = jnp.dot(patch, w_ref[t, cb],
                        preferred_element_type=jnp.float32)
            acc = d if acc is None else acc + d
            if (dy, dx) == (1, 1):
                dsc = jnp.dot(patch, wsc_ref[cb],
                              preferred_element_type=jnp.float32)
                asc = dsc if asc is None else asc + dsc
    y_ref[...] = acc.reshape(B, Ho, Wo, ct).astype(jnp.bfloat16)
    s_ref[0, 0:1, :] = jnp.sum(acc, axis=0, keepdims=True)
    s_ref[0, 1:2, :] = jnp.sum(acc * acc, axis=0, keepdims=True)
    ysc_ref[...] = asc.reshape(B, Ho, Wo, ct).astype(jnp.bfloat16)
    ssc_ref[0, 0:1, :] = jnp.sum(asc, axis=0, keepdims=True)
    ssc_ref[0, 1:2, :] = jnp.sum(asc * asc, axis=0, keepdims=True)


def _sconv_call(a, w, wsc, B):
    N, H, W, Cin = a.shape
    Ho, Wo = H // 2, W // 2
    Cout = w.shape[2]
    ct = Cout
    G = N // B
    CB = Cin // 128
    w4 = w.reshape(9, CB, 128, Cout)
    wsc3 = wsc.reshape(CB, 128, Cout)

    kern = functools.partial(_sconv_kernel, B=B, H=H, W=W, Cin=Cin, ct=ct)
    return pl.pallas_call(
        kern,
        out_shape=(jax.ShapeDtypeStruct((N, Ho, Wo, Cout), jnp.bfloat16),
                   jax.ShapeDtypeStruct((G, 2, Cout), jnp.float32),
                   jax.ShapeDtypeStruct((N, Ho, Wo, Cout), jnp.bfloat16),
                   jax.ShapeDtypeStruct((G, 2, Cout), jnp.float32)),
        grid=(G,),
        in_specs=[
            pl.BlockSpec((B, H, W, Cin), lambda n: (n, 0, 0, 0)),
            pl.BlockSpec((9, CB, 128, ct), lambda n: (0, 0, 0, 0)),
            pl.BlockSpec((CB, 128, ct), lambda n: (0, 0, 0)),
        ],
        out_specs=[pl.BlockSpec((B, Ho, Wo, ct), lambda n: (n, 0, 0, 0)),
                   pl.BlockSpec((1, 2, ct), lambda n: (n, 0, 0)),
                   pl.BlockSpec((B, Ho, Wo, ct), lambda n: (n, 0, 0, 0)),
                   pl.BlockSpec((1, 2, ct), lambda n: (n, 0, 0))],
        scratch_shapes=[pltpu.VMEM((B, H + 1, (W + 1) * CB, 128),
                                   jnp.float32)],
        compiler_params=pltpu.CompilerParams(
            dimension_semantics=("arbitrary",)),
    )(a, w4, wsc3)


def _gemm_kernel(x_ref, w_ref, y_ref, s_ref, *, B, M, K, ct):
    acc = jnp.dot(x_ref[...].reshape(B * M, K), w_ref[...],
                  preferred_element_type=jnp.float32)
    y_ref[...] = acc.reshape(B, M, ct).astype(jnp.bfloat16)
    s_ref[0, 0:1, :] = jnp.sum(acc, axis=0, keepdims=True)
    s_ref[0, 1:2, :] = jnp.sum(acc * acc, axis=0, keepdims=True)


def _gemm_call(x3, w2, B):
    N, M, K = x3.shape
    Cout = w2.shape[1]
    ct = Cout
    G = N // B
    kern = functools.partial(_gemm_kernel, B=B, M=M, K=K, ct=ct)
    return pl.pallas_call(
        kern,
        out_shape=(jax.ShapeDtypeStruct((N, M, Cout), jnp.bfloat16),
                   jax.ShapeDtypeStruct((G, 2, Cout), jnp.float32)),
        grid=(G, Cout // ct),
        in_specs=[pl.BlockSpec((B, M, K), lambda n, j: (n, 0, 0)),
                  pl.BlockSpec((K, ct), lambda n, j: (0, j))],
        out_specs=[pl.BlockSpec((B, M, ct), lambda n, j: (n, 0, j)),
                   pl.BlockSpec((1, 2, ct), lambda n, j: (n, 0, j))],
        compiler_params=pltpu.CompilerParams(
            dimension_semantics=("parallel", "arbitrary")),
    )(x3, w2)


def _blend_kernel(y_ref, st_ref, g_ref, b_ref, r_ref, o_ref, *, count, padc):
    scale, shift = _scale_shift(st_ref, g_ref, b_ref, count)
    y = y_ref[...].astype(jnp.float32) * scale + shift
    o = jnp.maximum(y + r_ref[...].astype(jnp.float32), 0.0)
    o_ref[...] = jnp.pad(o, ((0, 0), (0, padc))).astype(o_ref.dtype)


def _blend_pool_kernel(y_ref, st_ref, g_ref, b_ref, r_ref, o_ref, *, count,
                       rows):
    scale, shift = _scale_shift(st_ref, g_ref, b_ref, count)
    y = y_ref[...].astype(jnp.float32) * scale + shift
    o = jnp.maximum(y + r_ref[...].astype(jnp.float32), 0.0)
    ob = o.astype(jnp.bfloat16).astype(jnp.float32)
    tm, tc = y_ref.shape
    o_ref[...] = jnp.mean(ob.reshape(tm // rows, rows, tc), axis=1)


def _blend_call(y2d, st, g, b, r2d, pool_rows=0, pad_to=0):
    M, C = y2d.shape
    Co = max(C, pad_to)
    tm = 4096 if M % 4096 == 0 else (1024 if M % 1024 == 0 else M)
    tc = C
    count = float(M)
    slab = pl.BlockSpec((tm, tc), lambda i, j: (i, j))
    in_specs = [slab,
                pl.BlockSpec(st.shape, lambda i, j: (0, 0, 0)),
                pl.BlockSpec((1, tc), lambda i, j: (0, 0)),
                pl.BlockSpec((1, tc), lambda i, j: (0, 0)),
                slab]
    if pool_rows:
        tm = min(tm, M)
        kern = functools.partial(_blend_pool_kernel, count=count,
                                 rows=pool_rows)
        out_shape = jax.ShapeDtypeStruct((M // pool_rows, C), jnp.float32)
        out_specs = pl.BlockSpec((tm // pool_rows, tc), lambda i, j: (i, j))
    else:
        kern = functools.partial(_blend_kernel, count=count, padc=Co - C)
        out_shape = jax.ShapeDtypeStruct((M, Co), jnp.bfloat16)
        out_specs = pl.BlockSpec((tm, Co), lambda i, j: (i, j))
    return pl.pallas_call(
        kern,
        out_shape=out_shape,
        grid=(M // tm, C // tc),
        in_specs=in_specs,
        out_specs=out_specs,
        compiler_params=pltpu.CompilerParams(
            dimension_semantics=("parallel", "parallel")),
    )(y2d, st, g, b, r2d)


def _im2col_s1(x):
    N, H, W, C = x.shape
    xp = jnp.pad(x, ((0, 0), (1, 1), (1, 1), (0, 0)))
    cols = [xp[:, dy:dy + H, dx:dx + W, :] for dy, dx in _TAPS]
    return jnp.stack(cols, axis=3).reshape(N, H * W, 9 * C)


def kernel(x, conv1_w, bn1_g, bn1_b, layer1_0_conv1_w, layer1_0_conv2_w, layer1_0_bn1_g, layer1_0_bn1_b, layer1_0_bn2_g, layer1_0_bn2_b, layer1_1_conv1_w, layer1_1_conv2_w, layer1_1_bn1_g, layer1_1_bn1_b, layer1_1_bn2_g, layer1_1_bn2_b, layer2_0_conv1_w, layer2_0_conv2_w, layer2_0_bn1_g, layer2_0_bn1_b, layer2_0_bn2_g, layer2_0_bn2_b, layer2_0_sc_w, layer2_0_sc_g, layer2_0_sc_b, layer2_1_conv1_w, layer2_1_conv2_w, layer2_1_bn1_g, layer2_1_bn1_b, layer2_1_bn2_g, layer2_1_bn2_b, layer3_0_conv1_w, layer3_0_conv2_w, layer3_0_bn1_g, layer3_0_bn1_b, layer3_0_bn2_g, layer3_0_bn2_b, layer3_0_sc_w, layer3_0_sc_g, layer3_0_sc_b, layer3_1_conv1_w, layer3_1_conv2_w, layer3_1_bn1_g, layer3_1_bn1_b, layer3_1_bn2_g, layer3_1_bn2_b, layer4_0_conv1_w, layer4_0_conv2_w, layer4_0_bn1_g, layer4_0_bn1_b, layer4_0_bn2_g, layer4_0_bn2_b, layer4_0_sc_w, layer4_0_sc_g, layer4_0_sc_b, layer4_1_conv1_w, layer4_1_conv2_w, layer4_1_bn1_g, layer4_1_bn1_b, layer4_1_bn2_g, layer4_1_bn2_b, fc_w, fc_b):
    N = x.shape[0]
    xb = jnp.transpose(x, (0, 2, 3, 1)).astype(jnp.bfloat16)

    cols = _im2col_s1(xb)
    Cr = conv1_w.shape[2]
    y0, s0 = _gemm_call(cols, conv1_w.reshape(-1, conv1_w.shape[2])[:, :Cr],
                        B=min(4, N))
    y0 = y0.reshape(N, 32, 32, Cr)

    def cw(w):
        return w[:, :Cr, :Cr]

    def cv(v):
        return v[:, :Cr]

    y1, s1, h0 = _conv_call("bn", min(4, N), cw(layer1_0_conv1_w), y0,
                            bn1=(s0, cv(bn1_g), cv(bn1_b)), write_aux=True)
    y2, s2 = _conv_call("bn", min(4, N), cw(layer1_0_conv2_w), y1,
                        bn1=(s1, cv(layer1_0_bn1_g), cv(layer1_0_bn1_b)))
    y1b, s1b, a10 = _conv_call("bn_add", min(4, N), cw(layer1_1_conv1_w), y2,
                               bn1=(s2, cv(layer1_0_bn2_g),
                                    cv(layer1_0_bn2_b)),
                               resid=h0, write_aux=True)
    y2b, s2b = _conv_call("bn", min(4, N), cw(layer1_1_conv2_w), y1b,
                          bn1=(s1b, cv(layer1_1_bn1_g), cv(layer1_1_bn1_b)))
    C1 = layer1_1_conv2_w.shape[2]
    a1 = _blend_call(y2b.reshape(-1, Cr), s2b, cv(layer1_1_bn2_g),
                     cv(layer1_1_bn2_b), a10.reshape(-1, Cr),
                     pad_to=C1).reshape(N, 32, 32, C1)

    def down_stage(a, c1w, c2w, g1, b1, g2, b2, scw, scg, scb,
                   c1wB, c2wB, g1B, b1B, g2B, b2B, B, Bs, pool_rows=0):
        Ho = a.shape[1] // 2
        Cout = c1w.shape[2]
        y1, s1, ysc, ssc = _sconv_call(a, c1w, scw, Bs)
        y2, s2 = _conv_call("bn", B, c2w, y1, bn1=(s1, g1, b1))
        y1b, s1b, a0 = _conv_call("bn2", B, c1wB, y2, bn1=(s2, g2, b2),
                                  ysc=ysc, bn2=(ssc, scg, scb),
                                  write_aux=True)
        y2b, s2b = _conv_call("bn", B, c2wB, y1b, bn1=(s1b, g1B, b1B))
        out = _blend_call(y2b.reshape(-1, Cout), s2b, g2B, b2B,
                          a0.reshape(-1, Cout), pool_rows=pool_rows)
        if pool_rows:
            return out
        return out.reshape(N, Ho, Ho, Cout)

    a2 = down_stage(a1, layer2_0_conv1_w, layer2_0_conv2_w,
                    layer2_0_bn1_g, layer2_0_bn1_b, layer2_0_bn2_g,
                    layer2_0_bn2_b, layer2_0_sc_w, layer2_0_sc_g,
                    layer2_0_sc_b, layer2_1_conv1_w, layer2_1_conv2_w,
                    layer2_1_bn1_g, layer2_1_bn1_b, layer2_1_bn2_g,
                    layer2_1_bn2_b, B=min(8, N), Bs=min(4, N))
    a3 = down_stage(a2, layer3_0_conv1_w, layer3_0_conv2_w,
                    layer3_0_bn1_g, layer3_0_bn1_b, layer3_0_bn2_g,
                    layer3_0_bn2_b, layer3_0_sc_w, layer3_0_sc_g,
                    layer3_0_sc_b, layer3_1_conv1_w, layer3_1_conv2_w,
                    layer3_1_bn1_g, layer3_1_bn1_b, layer3_1_bn2_g,
                    layer3_1_bn2_b, B=min(16, N), Bs=min(8, N))
    pooled = down_stage(a3, layer4_0_conv1_w, layer4_0_conv2_w,
                        layer4_0_bn1_g, layer4_0_bn1_b, layer4_0_bn2_g,
                        layer4_0_bn2_b, layer4_0_sc_w, layer4_0_sc_g,
                        layer4_0_sc_b, layer4_1_conv1_w, layer4_1_conv2_w,
                        layer4_1_bn1_g, layer4_1_bn1_b, layer4_1_bn2_g,
                        layer4_1_bn2_b, B=min(32, N), Bs=min(16, N), pool_rows=16)

    return pooled @ fc_w + fc_b
